# baseline (speedup 1.0000x reference)
import functools

import jax
import jax.numpy as jnp
from jax import lax
from jax.experimental import pallas as pl
from jax.experimental.pallas import tpu as pltpu
from jax.experimental.pallas import tpu_sc as plsc

B = 16384
D = 4096
NUMC = 128
NC = 2
NS = 16
NW = NC * NS
W = D // NW
CHUNK = 256

BT = 8192
BSC = B - BT
NCHUNK = BSC // CHUNK
GPC = CHUNK // 16
BK = 1024
NBK = BT // BK


@functools.partial(
    pl.kernel,
    out_type=jax.ShapeDtypeStruct((NUMC, NW, W), jnp.float32),
    mesh=plsc.VectorSubcoreMesh(core_axis_name="c", subcore_axis_name="s"),
    compiler_params=pltpu.CompilerParams(needs_layout_passes=False),
    scratch_types=[
        pltpu.VMEM((GPC, 16), jnp.int32),
        pltpu.VMEM((GPC, 16), jnp.int32),
        pltpu.VMEM((CHUNK, W), jnp.float32),
        pltpu.VMEM((CHUNK, W), jnp.float32),
        pltpu.VMEM((NUMC, W), jnp.float32),
        pltpu.SemaphoreType.DMA,
        pltpu.SemaphoreType.DMA,
    ],
)
def _sc_segsum(hv_hbm, lab_hbm, out_hbm, lab0, lab1, buf0, buf1, acc_v,
               sem0, sem1):
    c = lax.axis_index("c")
    s = lax.axis_index("s")
    w = c * NS + s

    zero16 = jnp.zeros((16,), jnp.float32)

    def zv(i, _):
        for j in range(W // 16):
            acc_v[i, pl.ds(j * 16, 16)] = zero16
        return 0

    lax.fori_loop(0, NUMC, zv, 0)

    lanes = lax.iota(jnp.int32, 16)
    bufs = (buf0, buf1)
    labs = (lab0, lab1)
    sems = (sem0, sem1)
    _dnums = lax.GatherDimensionNumbers(
        offset_dims=(), collapsed_slice_dims=(0,), start_index_map=(0,))

    def lane_bcast(v, i):
        return lax.gather(v, jnp.full((16, 1), i, jnp.int32), _dnums, (1,),
                          mode=lax.GatherScatterMode.PROMISE_IN_BOUNDS)

    def data_src(cid):
        return hv_hbm.at[pl.ds(BT + cid * CHUNK, CHUNK), pl.ds(w * W, W)]

    def lab_src(cid):
        return lab_hbm.at[pl.ds(BT // 16 + cid * GPC, GPC)]

    def issue(cid, b):
        pltpu.async_copy(data_src(cid), bufs[b], sems[b])
        pltpu.async_copy(lab_src(cid), labs[b], sems[b])

    def drain(cid, b):
        pltpu.make_async_copy(data_src(cid), bufs[b], sems[b]).wait()
        pltpu.make_async_copy(lab_src(cid), labs[b], sems[b]).wait()

    colvecs = [lanes + (j * 16) for j in range(W // 16)]

    def compute(buf, lab_c):
        @plsc.parallel_loop(0, GPC)
        def group(g):
            lv = lab_c[g]

            for i in range(16):
                lsplat = lane_bcast(lv, i)
                r = g * 16 + i
                for j in range(W // 16):
                    x = buf[r, pl.ds(j * 16, 16)]
                    plsc.addupdate_scatter(acc_v, [lsplat, colvecs[j]], x)

    issue(0, 0)

    def ring(k2, _):
        for b in range(2):
            cid = k2 * 2 + b
            nid = cid + 1

            @pl.when(nid < NCHUNK)
            def _():
                issue(nid, 1 - b)

            drain(cid, b)
            compute(bufs[b], labs[b])
        return 0

    lax.fori_loop(0, NCHUNK // 2, ring, 0)

    pltpu.sync_copy(acc_v, out_hbm.at[pl.ds(0, NUMC), w])


def _tc_body(lab_ref, hv_ref, out_ref):
    k = pl.program_id(0)

    @pl.when(k == 0)
    def _():
        out_ref[...] = jnp.zeros_like(out_ref)

    lab = lab_ref[0, 0, :]
    onehot = (lax.broadcasted_iota(jnp.int32, (NUMC, BK), 0)
              == lab[None, :]).astype(jnp.float32)
    out_ref[...] += jnp.dot(onehot, hv_ref[...],
                            preferred_element_type=jnp.float32)


_tc_segsum = pl.pallas_call(
    _tc_body,
    out_shape=jax.ShapeDtypeStruct((NUMC, D), jnp.float32),
    grid=(NBK,),
    in_specs=[
        pl.BlockSpec((1, 1, BK), lambda k: (k, 0, 0)),
        pl.BlockSpec((BK, D), lambda k: (k, 0)),
    ],
    out_specs=pl.BlockSpec((NUMC, D), lambda k: (0, 0)),
)


def kernel(dataset_hvs, labels, num_classes):
    lab = (labels % num_classes).astype(jnp.int32)
    sc_out = _sc_segsum(dataset_hvs, lab.reshape(B // 16, 16))
    tc_out = _tc_segsum(lab.reshape(B // BK, 1, BK), dataset_hvs)
    return tc_out + sc_out.reshape(NUMC, D)


# speedup vs baseline: 1.4086x; 1.4086x over previous
import functools

import jax
import jax.numpy as jnp
from jax import lax
from jax.experimental import pallas as pl
from jax.experimental.pallas import tpu as pltpu
from jax.experimental.pallas import tpu_sc as plsc

B = 16384
D = 4096
NUMC = 128
NC = 2
NS = 16
NW = NC * NS
W = D // NW
CHUNK = 256

BT = 12288
BSC = B - BT
NCHUNK = BSC // CHUNK
GPC = CHUNK // 16
BK = 1024
NBK = BT // BK


@functools.partial(
    pl.kernel,
    out_type=jax.ShapeDtypeStruct((NUMC, NW, W), jnp.float32),
    mesh=plsc.VectorSubcoreMesh(core_axis_name="c", subcore_axis_name="s"),
    compiler_params=pltpu.CompilerParams(needs_layout_passes=False),
    scratch_types=[
        pltpu.VMEM((GPC, 16), jnp.int32),
        pltpu.VMEM((GPC, 16), jnp.int32),
        pltpu.VMEM((CHUNK, W), jnp.float32),
        pltpu.VMEM((CHUNK, W), jnp.float32),
        pltpu.VMEM((NUMC, W), jnp.float32),
        pltpu.SemaphoreType.DMA,
        pltpu.SemaphoreType.DMA,
    ],
)
def _sc_segsum(hv_hbm, lab_hbm, out_hbm, lab0, lab1, buf0, buf1, acc_v,
               sem0, sem1):
    c = lax.axis_index("c")
    s = lax.axis_index("s")
    w = c * NS + s

    zero16 = jnp.zeros((16,), jnp.float32)

    def zv(i, _):
        for j in range(W // 16):
            acc_v[i, pl.ds(j * 16, 16)] = zero16
        return 0

    lax.fori_loop(0, NUMC, zv, 0)

    lanes = lax.iota(jnp.int32, 16)
    bufs = (buf0, buf1)
    labs = (lab0, lab1)
    sems = (sem0, sem1)
    _dnums = lax.GatherDimensionNumbers(
        offset_dims=(), collapsed_slice_dims=(0,), start_index_map=(0,))

    def lane_bcast(v, i):
        return lax.gather(v, jnp.full((16, 1), i, jnp.int32), _dnums, (1,),
                          mode=lax.GatherScatterMode.PROMISE_IN_BOUNDS)

    def data_src(cid):
        return hv_hbm.at[pl.ds(BT + cid * CHUNK, CHUNK), pl.ds(w * W, W)]

    def lab_src(cid):
        return lab_hbm.at[pl.ds(BT // 16 + cid * GPC, GPC)]

    def issue(cid, b):
        pltpu.async_copy(data_src(cid), bufs[b], sems[b])
        pltpu.async_copy(lab_src(cid), labs[b], sems[b])

    def drain(cid, b):
        pltpu.make_async_copy(data_src(cid), bufs[b], sems[b]).wait()
        pltpu.make_async_copy(lab_src(cid), labs[b], sems[b]).wait()

    colvecs = [lanes + (j * 16) for j in range(W // 16)]

    def compute(buf, lab_c):
        @plsc.parallel_loop(0, GPC)
        def group(g):
            lv = lab_c[g]

            for i in range(16):
                lsplat = lane_bcast(lv, i)
                r = g * 16 + i
                for j in range(W // 16):
                    x = buf[r, pl.ds(j * 16, 16)]
                    plsc.addupdate_scatter(acc_v, [lsplat, colvecs[j]], x)

    issue(0, 0)

    def ring(k2, _):
        for b in range(2):
            cid = k2 * 2 + b
            nid = cid + 1

            @pl.when(nid < NCHUNK)
            def _():
                issue(nid, 1 - b)

            drain(cid, b)
            compute(bufs[b], labs[b])
        return 0

    lax.fori_loop(0, NCHUNK // 2, ring, 0)

    pltpu.sync_copy(acc_v, out_hbm.at[pl.ds(0, NUMC), w])


def _tc_body(lab_ref, hv_ref, out_ref):
    k = pl.program_id(0)

    @pl.when(k == 0)
    def _():
        out_ref[...] = jnp.zeros_like(out_ref)

    lab = lab_ref[0, 0, :]
    onehot = (lax.broadcasted_iota(jnp.int32, (NUMC, BK), 0)
              == lab[None, :]).astype(jnp.float32)
    out_ref[...] += jnp.dot(onehot, hv_ref[...],
                            preferred_element_type=jnp.float32)


_tc_segsum = pl.pallas_call(
    _tc_body,
    out_shape=jax.ShapeDtypeStruct((NUMC, D), jnp.float32),
    grid=(NBK,),
    in_specs=[
        pl.BlockSpec((1, 1, BK), lambda k: (k, 0, 0)),
        pl.BlockSpec((BK, D), lambda k: (k, 0)),
    ],
    out_specs=pl.BlockSpec((NUMC, D), lambda k: (0, 0)),
)


def kernel(dataset_hvs, labels, num_classes):
    lab = (labels % num_classes).astype(jnp.int32)
    sc_out = _sc_segsum(dataset_hvs, lab.reshape(B // 16, 16))
    tc_out = _tc_segsum(lab.reshape(B // BK, 1, BK), dataset_hvs)
    return tc_out + sc_out.reshape(NUMC, D)
